# baseline (device time: 750440 ns/iter reference)
import jax
import jax.numpy as jnp
from jax import lax
from jax.experimental import pallas as pl
from jax.experimental.pallas import tpu as pltpu

N_DEV = 4
M = 4096
K = 4096
N = 8192
KS = K // N_DEV
HS = KS // 2
NB = 1024
N_BLOCKS = N // NB
N_ACC = 2
MT = 512


def _fused_body(s_ref, x_hbm, w_hbm, o_ref, wg_ref,
                xs, wwave, acc, wtail, stage,
                send_s, recv_s, copy_sem, store_sem):
    my = lax.axis_index("i")
    left = lax.rem(my + N_DEV - 1, N_DEV)
    right = lax.rem(my + 1, N_DEV)
    scale = s_ref[0, 0]

    def _copy(src, dst):
        c = pltpu.make_async_copy(src, dst, copy_sem)
        c.start()
        c.wait()

    _copy(x_hbm, xs.at[:, pl.ds(0, KS)])
    _copy(w_hbm, wg_ref.at[pl.ds(0, KS), :])
    _copy(w_hbm.at[:, pl.ds(0, N_ACC * NB)], wwave)

    barrier = pltpu.get_barrier_semaphore()
    for nbr in (left, right):
        pl.semaphore_signal(barrier, inc=1, device_id=(nbr,),
                            device_id_type=pl.DeviceIdType.MESH)
    pl.semaphore_wait(barrier, 2)

    def seg(h, is_left):
        return pl.ds(h * KS + (HS if is_left else 0), HS)

    send_descs = []

    def flows(h):
        for f, (is_x, is_left) in enumerate(
                [(True, False), (True, True), (False, False), (False, True)]):
            if is_x:
                src = xs.at[:, seg(h, is_left)]
                dst = xs.at[:, seg(h + 1, is_left)]
            else:
                src = wg_ref.at[seg(h, is_left), :]
                dst = wg_ref.at[seg(h + 1, is_left), :]
            yield f, src, dst, is_left

    def send_hop(h):
        for f, src, dst, is_left in flows(h):
            d = pltpu.make_async_remote_copy(
                src_ref=src, dst_ref=dst,
                send_sem=send_s.at[f, h], recv_sem=recv_s.at[f, h],
                device_id=(left if is_left else right,),
                device_id_type=pl.DeviceIdType.MESH)
            d.start()
            send_descs.append(d)

    def recv_hop(h):
        for f, src, dst, is_left in flows(h):
            d = pltpu.make_async_remote_copy(
                src_ref=dst, dst_ref=dst,
                send_sem=send_s.at[f, h], recv_sem=recv_s.at[f, h],
                device_id=(right if is_left else left,),
                device_id_type=pl.DeviceIdType.MESH)
            d.wait_recv()

    def wave_compute(wi, init):
        ksl = pl.ds(wi * KS, KS)
        for b in range(N_ACC):
            for mi in range(M // MT):
                msl = pl.ds(mi * MT, MT)
                r = jnp.dot(xs[msl, ksl], wwave[:, pl.ds(b * NB, NB)],
                            preferred_element_type=jnp.float32)
                if not init:
                    r = acc[b, msl, :].astype(jnp.float32) + r
                acc[b, msl, :] = r.astype(jnp.bfloat16)

    send_hop(0)
    wave_compute(0, init=True)
    for h in range(N_DEV - 1):
        recv_hop(h)
        if h < N_DEV - 2:
            send_hop(h + 1)
        _copy(wg_ref.at[pl.ds((h + 1) * KS, KS), pl.ds(0, N_ACC * NB)],
              wwave)
        wave_compute(h + 1, init=False)

    pending = [None, None]
    tile_counter = [0]

    def write_tile(mi, b, val):
        p = tile_counter[0] % 2
        tile_counter[0] += 1
        if pending[p] is not None:
            pending[p].wait()
        stage[p, :, :] = val
        c = pltpu.make_async_copy(
            stage.at[p],
            o_ref.at[pl.ds(mi * MT, MT), pl.ds(b * NB, NB)],
            store_sem.at[p])
        c.start()
        pending[p] = c

    for b in range(N_ACC):
        for mi in range(M // MT):
            msl = pl.ds(mi * MT, MT)
            write_tile(mi, b, acc[b, msl, :].astype(jnp.float32) * scale)

    for b in range(N_ACC, N_BLOCKS):
        _copy(wg_ref.at[:, pl.ds(b * NB, NB)], wtail)
        for mi in range(M // MT):
            msl = pl.ds(mi * MT, MT)
            write_tile(mi, b, jnp.dot(
                xs[msl, :], wtail[...],
                preferred_element_type=jnp.float32) * scale)

    for p in range(2):
        if pending[p] is not None:
            pending[p].wait()
    for d in send_descs:
        d.wait_send()


def kernel(x, w_mat, scale_x, scale_w):
    xc = x.astype(jnp.float8_e4m3fn)
    wc = w_mat.astype(jnp.float8_e5m2)
    s = (scale_x * scale_w).reshape(1, 1)
    dma43 = pltpu.SemaphoreType.DMA((4, N_DEV - 1))
    out, _wg = pl.pallas_call(
        _fused_body,
        out_shape=[
            jax.ShapeDtypeStruct((M, N), jnp.float32),
            jax.ShapeDtypeStruct((K, N), jnp.float8_e5m2),
        ],
        in_specs=[
            pl.BlockSpec(memory_space=pltpu.SMEM),
            pl.BlockSpec(memory_space=pl.ANY),
            pl.BlockSpec(memory_space=pl.ANY),
        ],
        out_specs=[pl.BlockSpec(memory_space=pl.ANY),
                   pl.BlockSpec(memory_space=pl.ANY)],
        scratch_shapes=[
            pltpu.VMEM((M, K), jnp.float8_e4m3fn),
            pltpu.VMEM((KS, N_ACC * NB), jnp.float8_e5m2),
            pltpu.VMEM((N_ACC, M, NB), jnp.bfloat16),
            pltpu.VMEM((K, NB), jnp.float8_e5m2),
            pltpu.VMEM((2, MT, NB), jnp.float32),
            dma43,
            dma43,
            pltpu.SemaphoreType.DMA,
            pltpu.SemaphoreType.DMA((N_ACC,)),
        ],
        compiler_params=pltpu.CompilerParams(
            collective_id=0,
            vmem_limit_bytes=60 * 1024 * 1024,
        ),
    )(s, xc, wc)
    return out


# device time: 385665 ns/iter; 1.9458x vs baseline; 1.9458x over previous
import jax
import jax.numpy as jnp
from jax import lax
from jax.experimental import pallas as pl
from jax.experimental.pallas import tpu as pltpu

N_DEV = 4
KS = 1024
HS = KS // 2

BM = 1024
BN = 2048


def _ag_body(x_hbm, w_hbm, xg_ref, wg_ref, send_s, recv_s, copy_sem):
    my = lax.axis_index("i")
    left = lax.rem(my + N_DEV - 1, N_DEV)
    right = lax.rem(my + 1, N_DEV)

    cx = pltpu.make_async_copy(x_hbm, xg_ref.at[:, pl.ds(0, KS)], copy_sem)
    cx.start()
    cw = pltpu.make_async_copy(w_hbm, wg_ref.at[pl.ds(0, KS), :], copy_sem)
    cw.start()

    barrier = pltpu.get_barrier_semaphore()
    for nbr in (left, right):
        pl.semaphore_signal(barrier, inc=1, device_id=(nbr,),
                            device_id_type=pl.DeviceIdType.MESH)
    pl.semaphore_wait(barrier, 2)
    cx.wait()
    cw.wait()

    def seg(h, is_left):
        return pl.ds(h * KS + (HS if is_left else 0), HS)

    def flows(h):
        for f, (is_x, is_left) in enumerate(
                [(True, False), (True, True), (False, False), (False, True)]):
            if is_x:
                src = xg_ref.at[:, seg(h, is_left)]
                dst = xg_ref.at[:, seg(h + 1, is_left)]
            else:
                src = wg_ref.at[seg(h, is_left), :]
                dst = wg_ref.at[seg(h + 1, is_left), :]
            yield f, src, dst, is_left

    send_descs = []

    def send_hop(h):
        for f, src, dst, is_left in flows(h):
            d = pltpu.make_async_remote_copy(
                src_ref=src, dst_ref=dst,
                send_sem=send_s.at[f, h], recv_sem=recv_s.at[f, h],
                device_id=(left if is_left else right,),
                device_id_type=pl.DeviceIdType.MESH)
            d.start()
            send_descs.append(d)

    def recv_hop(h):
        for f, src, dst, is_left in flows(h):
            d = pltpu.make_async_remote_copy(
                src_ref=dst, dst_ref=dst,
                send_sem=send_s.at[f, h], recv_sem=recv_s.at[f, h],
                device_id=(right if is_left else left,),
                device_id_type=pl.DeviceIdType.MESH)
            d.wait_recv()

    send_hop(0)
    for h in range(N_DEV - 1):
        recv_hop(h)
        if h < N_DEV - 2:
            send_hop(h + 1)
    for d in send_descs:
        d.wait_send()


def _all_gather(xc, wc):
    m, kx = xc.shape
    kw, n = wc.shape
    dma43 = pltpu.SemaphoreType.DMA((4, N_DEV - 1))
    return pl.pallas_call(
        _ag_body,
        out_shape=[
            jax.ShapeDtypeStruct((m, N_DEV * kx), xc.dtype),
            jax.ShapeDtypeStruct((N_DEV * kw, n), wc.dtype),
        ],
        in_specs=[pl.BlockSpec(memory_space=pl.ANY),
                  pl.BlockSpec(memory_space=pl.ANY)],
        out_specs=[pl.BlockSpec(memory_space=pl.ANY),
                   pl.BlockSpec(memory_space=pl.ANY)],
        scratch_shapes=[dma43, dma43, pltpu.SemaphoreType.DMA],
        compiler_params=pltpu.CompilerParams(collective_id=0),
    )(xc, wc)


def _gemm_body(s_ref, x_ref, w_ref, o_ref):
    o_ref[...] = (
        jnp.dot(x_ref[...], w_ref[...], preferred_element_type=jnp.float32)
        * s_ref[0, 0]
    )


def _gemm(s, xg, wg):
    m, k = xg.shape
    _, n = wg.shape
    return pl.pallas_call(
        _gemm_body,
        grid=(n // BN, m // BM),
        in_specs=[
            pl.BlockSpec((1, 1), lambda j, i: (0, 0),
                         memory_space=pltpu.SMEM),
            pl.BlockSpec((BM, k), lambda j, i: (i, 0)),
            pl.BlockSpec((k, BN), lambda j, i: (0, j)),
        ],
        out_specs=pl.BlockSpec((BM, BN), lambda j, i: (i, j)),
        out_shape=jax.ShapeDtypeStruct((m, n), jnp.float32),
        compiler_params=pltpu.CompilerParams(
            dimension_semantics=("parallel", "parallel"),
            vmem_limit_bytes=56 * 1024 * 1024,
        ),
    )(s, xg, wg)


def kernel(x, w_mat, scale_x, scale_w):
    xc = x.astype(jnp.float8_e4m3fn)
    wc = w_mat.astype(jnp.float8_e5m2)
    xg, wg = _all_gather(xc, wc)
    s = (scale_x * scale_w).reshape(1, 1)
    return _gemm(s, xg, wg)


# device time: 385437 ns/iter; 1.9470x vs baseline; 1.0006x over previous
import jax
import jax.numpy as jnp
from jax import lax
from jax.experimental import pallas as pl
from jax.experimental.pallas import tpu as pltpu

N_DEV = 4
KS = 1024
HS = KS // 2

BM = 1024
BN = 2048


def _ag_body(x_hbm, w_hbm, xg_ref, wg_ref, send_s, recv_s, copy_sem):
    my = lax.axis_index("i")
    left = lax.rem(my + N_DEV - 1, N_DEV)
    right = lax.rem(my + 1, N_DEV)

    cx = pltpu.make_async_copy(x_hbm, xg_ref.at[:, pl.ds(0, KS)], copy_sem)
    cx.start()
    cw = pltpu.make_async_copy(w_hbm, wg_ref.at[pl.ds(0, KS), :], copy_sem)
    cw.start()

    barrier = pltpu.get_barrier_semaphore()
    for nbr in (left, right):
        pl.semaphore_signal(barrier, inc=1, device_id=(nbr,),
                            device_id_type=pl.DeviceIdType.MESH)
    pl.semaphore_wait(barrier, 2)
    cx.wait()
    cw.wait()

    def seg(h, is_left):
        return pl.ds(h * KS + (HS if is_left else 0), HS)

    def flows(h):
        for f, (is_x, is_left) in enumerate(
                [(True, False), (True, True), (False, False), (False, True)]):
            if is_x:
                src = xg_ref.at[:, seg(h, is_left)]
                dst = xg_ref.at[:, seg(h + 1, is_left)]
            else:
                src = wg_ref.at[seg(h, is_left), :]
                dst = wg_ref.at[seg(h + 1, is_left), :]
            yield f, src, dst, is_left

    send_descs = []

    def send_hop(h):
        for f, src, dst, is_left in flows(h):
            d = pltpu.make_async_remote_copy(
                src_ref=src, dst_ref=dst,
                send_sem=send_s.at[f, h], recv_sem=recv_s.at[f, h],
                device_id=(left if is_left else right,),
                device_id_type=pl.DeviceIdType.MESH)
            d.start()
            send_descs.append(d)

    def recv_hop(h):
        for f, src, dst, is_left in flows(h):
            d = pltpu.make_async_remote_copy(
                src_ref=dst, dst_ref=dst,
                send_sem=send_s.at[f, h], recv_sem=recv_s.at[f, h],
                device_id=(right if is_left else left,),
                device_id_type=pl.DeviceIdType.MESH)
            d.wait_recv()

    send_hop(0)
    for h in range(N_DEV - 1):
        recv_hop(h)
        if h < N_DEV - 2:
            send_hop(h + 1)
    for d in send_descs:
        d.wait_send()


def _all_gather(xc, wc):
    m, kx = xc.shape
    kw, n = wc.shape
    dma43 = pltpu.SemaphoreType.DMA((4, N_DEV - 1))
    return pl.pallas_call(
        _ag_body,
        out_shape=[
            jax.ShapeDtypeStruct((m, N_DEV * kx), xc.dtype),
            jax.ShapeDtypeStruct((N_DEV * kw, n), wc.dtype),
        ],
        in_specs=[pl.BlockSpec(memory_space=pl.ANY),
                  pl.BlockSpec(memory_space=pl.ANY)],
        out_specs=[pl.BlockSpec(memory_space=pl.ANY),
                   pl.BlockSpec(memory_space=pl.ANY)],
        scratch_shapes=[dma43, dma43, pltpu.SemaphoreType.DMA],
        compiler_params=pltpu.CompilerParams(collective_id=0),
    )(xc, wc)


def _gemm_body(s_ref, x_ref, w_ref, o_ref):
    o_ref[...] = (
        jnp.dot(x_ref[...], w_ref[...], preferred_element_type=jnp.float32)
        * s_ref[0, 0]
    )


def _gemm(s, xg, wg):
    m, k = xg.shape
    _, n = wg.shape
    return pl.pallas_call(
        _gemm_body,
        grid=(n // BN, m // BM),
        in_specs=[
            pl.BlockSpec((1, 1), lambda j, i: (0, 0),
                         memory_space=pltpu.SMEM),
            pl.BlockSpec((BM, k), lambda j, i: (i, 0)),
            pl.BlockSpec((k, BN), lambda j, i: (0, j)),
        ],
        out_specs=pl.BlockSpec((BM, BN), lambda j, i: (i, j)),
        out_shape=jax.ShapeDtypeStruct((m, n), jnp.float32),
        compiler_params=pltpu.CompilerParams(
            dimension_semantics=("parallel", "parallel"),
            vmem_limit_bytes=56 * 1024 * 1024,
        ),
    )(s, xg, wg)


def kernel(x, w_mat, scale_x, scale_w):
    xc = x.astype(jnp.float8_e4m3fn)
    wc = w_mat.astype(jnp.float8_e4m3fn)
    xg, wg = _all_gather(xc, wc)
    s = (scale_x * scale_w).reshape(1, 1)
    return _gemm(s, xg, wg)
